# baseline (speedup 1.0000x reference)
import functools

import jax
import jax.numpy as jnp
from jax import lax
from jax.experimental import pallas as pl
from jax.experimental.pallas import tpu as pltpu
from jax.experimental.pallas import tpu_sc as plsc

N_NODES = 40962
N_OUT = 10242
K = 7
NUM_WORKERS = 32
SLAB = 8
NODES_PER_W = N_OUT // NUM_WORKERS
GROUPS_PER_W = NODES_PER_W // 4
N_CHUNKS = N_OUT // 2
IDX_ROWS = ((N_CHUNKS + 7) // 8) * 8
X_ROWS = N_NODES * SLAB
O_ROWS = N_OUT * SLAB
SCALE = 1.0 / K
RB = (0, 56, 128, 184)


@functools.partial(
    pl.kernel,
    mesh=plsc.VectorSubcoreMesh(core_axis_name="c", subcore_axis_name="s"),
    compiler_params=pltpu.CompilerParams(needs_layout_passes=False),
    out_type=jax.ShapeDtypeStruct((O_ROWS, 128), jnp.float32),
    scratch_types=[
        pltpu.VMEM((168 * 128,), jnp.int32),
        pltpu.VMEM((256, 128), jnp.float32),
        pltpu.VMEM((256, 128), jnp.float32),
        pltpu.VMEM((32, 128), jnp.float32),
        pltpu.VMEM((32, 128), jnp.float32),
        pltpu.SemaphoreType.DMA,
        pltpu.SemaphoreType.DMA,
        pltpu.SemaphoreType.DMA,
        pltpu.SemaphoreType.DMA,
    ],
)
def _pool(x_hbm, idx_hbm, out_hbm, idx_v, gb0, gb1, st0, st1,
          sem_g0, sem_g1, sem_o0, sem_o1):
    wid = lax.axis_index("s") * 2 + lax.axis_index("c")
    pltpu.sync_copy(idx_hbm.at[pl.ds(wid * 20480, 20480)],
                    idx_v.at[pl.ds(0, 20480)])
    scale = jnp.float32(SCALE)
    gbufs = (gb0, gb1)
    gsems = (sem_g0, sem_g1)
    stages = (st0, st1)
    osems = (sem_o0, sem_o1)
    obase = wid * NODES_PER_W * SLAB

    def fire(g, p):
        for s in range(8):
            pltpu.async_copy(
                x_hbm.at[idx_v.at[pl.ds(g * 256 + 32 * s, 32)]],
                gbufs[p].at[pl.ds(32 * s, 32)], gsems[p])

    def drain(g, p):
        pltpu.make_async_copy(
            x_hbm.at[idx_v.at[pl.ds(g * 256, 256)]], gbufs[p], gsems[p]
        ).wait()

    def compute4(gb, st):
        def rr_step(rr, c2):
            for v in range(4):
                for m in range(8):
                    acc = gb[RB[v] + rr, pl.ds(16 * m, 16)]
                    for k in range(1, K):
                        acc = acc + gb[RB[v] + 8 * k + rr, pl.ds(16 * m, 16)]
                    st[8 * v + rr, pl.ds(16 * m, 16)] = acc * scale
            return c2
        lax.fori_loop(0, SLAB, rr_step, 0)

    fire(0, 0)
    fire(1, 1)

    def body(i, carry):
        for p in range(2):
            g = 2 * i + p
            drain(g, p)

            @pl.when(i > 0)
            def _():
                pltpu.make_async_copy(
                    stages[p], out_hbm.at[pl.ds(obase, 32)], osems[p]).wait()

            compute4(gbufs[p], stages[p])

            @pl.when(g + 2 < GROUPS_PER_W)
            def _():
                fire(g + 2, p)

            pltpu.async_copy(
                stages[p], out_hbm.at[pl.ds(obase + 32 * g, 32)], osems[p])
        return carry

    lax.fori_loop(0, GROUPS_PER_W // 2, body, 0)
    pltpu.make_async_copy(
        stages[0], out_hbm.at[pl.ds(obase, 32)], sem_o0).wait()
    pltpu.make_async_copy(
        stages[1], out_hbm.at[pl.ds(obase, 32)], sem_o1).wait()

    @pl.when(wid == NUM_WORKERS - 1)
    def _():
        pltpu.sync_copy(idx_hbm.at[pl.ds(5120 * 128, 1024)],
                        idx_v.at[pl.ds(20480, 1024)])
        for s in range(4):
            pltpu.async_copy(
                x_hbm.at[idx_v.at[pl.ds(20480 + 32 * s, 32)]],
                gb0.at[pl.ds(32 * s, 32)], sem_g0)
        pltpu.make_async_copy(
            x_hbm.at[idx_v.at[pl.ds(20480, 128)]],
            gb0.at[pl.ds(0, 128)], sem_g0).wait()

        def rr_step(rr, c2):
            for v in range(2):
                for m in range(8):
                    acc = gb0[56 * v + rr, pl.ds(16 * m, 16)]
                    for k in range(1, K):
                        acc = acc + gb0[56 * v + 8 * k + rr, pl.ds(16 * m, 16)]
                    st0[8 * v + rr, pl.ds(16 * m, 16)] = acc * scale
            return c2
        lax.fori_loop(0, SLAB, rr_step, 0)
        pltpu.sync_copy(st0.at[pl.ds(0, 16)],
                        out_hbm.at[pl.ds(O_ROWS - 16, 16)])


def kernel(x, neigh_orders):
    B, D, N = x.shape
    ne = neigh_orders[: N_OUT * K].astype(jnp.int32).reshape(N_OUT, K)
    e = (ne[:, :, None] * SLAB + jnp.arange(SLAB, dtype=jnp.int32))
    e = e.reshape(N_CHUNKS, 112)
    e = jnp.pad(e, ((0, IDX_ROWS - N_CHUNKS), (0, 16))).reshape(-1)
    xa = x.reshape(B, 2, 128, N).transpose(3, 1, 0, 2).reshape(X_ROWS, 128)
    out = _pool(xa, e)
    out = out.reshape(N_OUT, 2, B, 128).transpose(2, 1, 3, 0)
    return out.reshape(B, D, N_OUT)
